# baseline (speedup 1.0000x reference)
import functools

import jax
import jax.numpy as jnp
from jax import lax
from jax.experimental import pallas as pl
from jax.experimental.pallas import tpu as pltpu
from jax.experimental.pallas import tpu_sc as plsc

NC = 2
NS = 16
NW = NC * NS

NJ = 26
NI = 16384
D = 32
B = NI * NJ

RPW = B // NW
GR = 1024
NG = RPW // GR

mesh = plsc.VectorSubcoreMesh(core_axis_name="c", subcore_axis_name="s")


@functools.partial(
    pl.kernel,
    mesh=mesh,
    out_type=jax.ShapeDtypeStruct((B, D), jnp.float32),
    scratch_types=[
        pltpu.VMEM((RPW,), jnp.int32),
        pltpu.VMEM((GR, D), jnp.float32),
        pltpu.VMEM((GR, D), jnp.float32),
        pltpu.SemaphoreType.DMA,
        pltpu.SemaphoreType.DMA,
        pltpu.SemaphoreType.DMA,
        pltpu.SemaphoreType.DMA,
    ],
    compiler_params=pltpu.CompilerParams(use_tc_tiling_on_sc=False),
)
def _gather(idxf, wt, inter, ixv, g0, g1, sg0, sg1, sw0, sw1):
    wid = lax.axis_index("s") * NC + lax.axis_index("c")
    base = wid * RPW
    pltpu.sync_copy(idxf.at[pl.ds(base, RPW)], ixv)

    def g_copy(g, gv, sem):
        return pltpu.make_async_copy(
            wt.at[ixv.at[pl.ds(g * GR, GR)]], gv, sem
        )

    def w_copy(g, gv, sem):
        return pltpu.make_async_copy(
            gv, inter.at[pl.ds(base + g * GR, GR)], sem
        )

    g_copy(0, g0, sg0).start()

    def body(g, _):
        @pl.when(lax.rem(g, 2) == 0)
        def _():
            @pl.when(g + 1 < NG)
            def _():
                @pl.when(g >= 1)
                def _():
                    w_copy(g - 1, g1, sw1).wait()

                g_copy(g + 1, g1, sg1).start()

            g_copy(g, g0, sg0).wait()
            w_copy(g, g0, sw0).start()

        @pl.when(lax.rem(g, 2) == 1)
        def _():
            @pl.when(g + 1 < NG)
            def _():
                w_copy(g - 1, g0, sw0).wait()
                g_copy(g + 1, g0, sg0).start()

            g_copy(g, g1, sg1).wait()
            w_copy(g, g1, sw1).start()

        return ()

    lax.fori_loop(0, NG, body, ())
    w_copy(NG - 2, g0, sw0).wait()
    w_copy(NG - 1, g1, sw1).wait()


V = 1000000
WBR = 2000


def _tcw_body(in_ref, out_ref):
    out_ref[...] = in_ref[...].reshape(WBR * 128)


_wlaunder = pl.pallas_call(
    _tcw_body,
    out_shape=jax.ShapeDtypeStruct((V * D,), jnp.float32),
    grid=(V * D // (WBR * 128),),
    in_specs=[pl.BlockSpec((WBR, 128), lambda t: (t, 0))],
    out_specs=pl.BlockSpec((WBR * 128,), lambda t: (t,)),
)


RCH = 512
NCH = NI // (4 * RCH)


def _tc_body(in_ref, out_ref):
    x = in_ref[...]
    xt = x.T.reshape(4, D, RCH)
    out_ref[...] = xt.transpose(1, 0, 2).reshape(1, D, 4 * RCH)


_transpose = pl.pallas_call(
    _tc_body,
    out_shape=jax.ShapeDtypeStruct((NJ, D, NI), jnp.float32),
    grid=(NJ, NCH),
    in_specs=[pl.BlockSpec((RCH, 128), lambda j, t: (j * NCH + t, 0))],
    out_specs=pl.BlockSpec((1, D, 4 * RCH), lambda j, t: (j, 0, t)),
)


def kernel(idx, weight):
    idxp = (
        idx.T.reshape(NJ, NCH, 4, RCH)
        .transpose(0, 1, 3, 2)
        .reshape(-1)
        .astype(jnp.int32)
    )
    wt = _wlaunder(weight.reshape(V * D // 128, 128)).reshape(V, D)
    inter = _gather(idxp, wt)
    o3 = _transpose(inter.reshape(B // 4, 128))
    return o3.transpose(2, 0, 1)


# speedup vs baseline: 1.1086x; 1.1086x over previous
import functools

import jax
import jax.numpy as jnp
from jax import lax
from jax.experimental import pallas as pl
from jax.experimental.pallas import tpu as pltpu
from jax.experimental.pallas import tpu_sc as plsc

NC = 2
NS = 16
NW = NC * NS

NJ = 26
NI = 16384
D = 32
B = NI * NJ

mesh = plsc.VectorSubcoreMesh(core_axis_name="c", subcore_axis_name="s")


def _make_gather(nrows, gr):
  rpw = nrows // NW
  ng = rpw // gr

  @functools.partial(
      pl.kernel,
      mesh=mesh,
      out_type=jax.ShapeDtypeStruct((nrows, D), jnp.float32),
      scratch_types=[
          pltpu.VMEM((rpw,), jnp.int32),
          pltpu.VMEM((gr, D), jnp.float32),
          pltpu.VMEM((gr, D), jnp.float32),
          pltpu.SemaphoreType.DMA,
          pltpu.SemaphoreType.DMA,
          pltpu.SemaphoreType.DMA,
          pltpu.SemaphoreType.DMA,
      ],
      compiler_params=pltpu.CompilerParams(use_tc_tiling_on_sc=False),
  )
  def _gather(idxf, wt, inter, ixv, g0, g1, sg0, sg1, sw0, sw1):
    RPW, GR, NG = rpw, gr, ng
    wid = lax.axis_index("s") * NC + lax.axis_index("c")
    base = wid * RPW
    pltpu.sync_copy(idxf.at[pl.ds(base, RPW)], ixv)

    def g_copy(g, gv, sem):
        return pltpu.make_async_copy(
            wt.at[ixv.at[pl.ds(g * GR, GR)]], gv, sem
        )

    def w_copy(g, gv, sem):
        return pltpu.make_async_copy(
            gv, inter.at[pl.ds(base + g * GR, GR)], sem
        )

    g_copy(0, g0, sg0).start()

    def body(g, _):
        @pl.when(lax.rem(g, 2) == 0)
        def _():
            @pl.when(g + 1 < NG)
            def _():
                @pl.when(g >= 1)
                def _():
                    w_copy(g - 1, g1, sw1).wait()

                g_copy(g + 1, g1, sg1).start()

            g_copy(g, g0, sg0).wait()
            w_copy(g, g0, sw0).start()

        @pl.when(lax.rem(g, 2) == 1)
        def _():
            @pl.when(g + 1 < NG)
            def _():
                w_copy(g - 1, g0, sw0).wait()
                g_copy(g + 1, g0, sg0).start()

            g_copy(g, g1, sg1).wait()
            w_copy(g, g1, sw1).start()

        return ()

    lax.fori_loop(0, NG, body, ())
    w_copy(NG - 2, g0, sw0).wait()
    w_copy(NG - 1, g1, sw1).wait()

  return _gather


RCH = 512
NCH = NI // (4 * RCH)


def _tc_body(in_ref, out_ref):
    x = in_ref[...]
    xt = x.T.reshape(4, D, RCH)
    out_ref[...] = xt.transpose(1, 0, 2).reshape(1, D, 4 * RCH)


def _tc_body_alias(in_ref, alias_ref, out_ref):
    del alias_ref
    _tc_body(in_ref, out_ref)


JH = NJ // 2


_transpose_a = pl.pallas_call(
    _tc_body,
    out_shape=jax.ShapeDtypeStruct((NJ, D, NI), jnp.float32),
    grid=(JH, NCH),
    in_specs=[pl.BlockSpec((RCH, 128), lambda j, t: (j * NCH + t, 0))],
    out_specs=pl.BlockSpec((1, D, 4 * RCH), lambda j, t: (j, 0, t)),
)

_transpose_b = pl.pallas_call(
    _tc_body_alias,
    out_shape=jax.ShapeDtypeStruct((NJ, D, NI), jnp.float32),
    grid=(JH, NCH),
    in_specs=[
        pl.BlockSpec((RCH, 128), lambda j, t: (j * NCH + t, 0)),
        pl.BlockSpec(memory_space=pl.ANY),
    ],
    out_specs=pl.BlockSpec((1, D, 4 * RCH), lambda j, t: (j + JH, 0, t)),
    input_output_aliases={1: 0},
)

_gather_half = _make_gather(B // 2, 832)


def kernel(idx, weight):
    idxp = (
        idx.T.reshape(2, JH, NCH, 4, RCH)
        .transpose(0, 1, 2, 4, 3)
        .reshape(2, B // 2)
        .astype(jnp.int32)
    )
    i1 = _gather_half(idxp[0], weight)
    i2 = _gather_half(idxp[1], weight)
    o3 = _transpose_a(i1.reshape(B // 8, 128))
    o3 = _transpose_b(i2.reshape(B // 8, 128), o3)
    return o3.transpose(2, 0, 1)


# speedup vs baseline: 1.1798x; 1.0642x over previous
import functools

import jax
import jax.numpy as jnp
from jax import lax
from jax.experimental import pallas as pl
from jax.experimental.pallas import tpu as pltpu
from jax.experimental.pallas import tpu_sc as plsc

NC = 2
NS = 16
NW = NC * NS

NJ = 26
NI = 16384
D = 32
B = NI * NJ

RPW = B // NW
GR = 1024
NG = RPW // GR

mesh = plsc.VectorSubcoreMesh(core_axis_name="c", subcore_axis_name="s")


@functools.partial(
    pl.kernel,
    mesh=mesh,
    out_type=jax.ShapeDtypeStruct((B, D), jnp.float32),
    scratch_types=[
        pltpu.VMEM((RPW,), jnp.int32),
        pltpu.VMEM((GR, D), jnp.float32),
        pltpu.VMEM((GR, D), jnp.float32),
        pltpu.SemaphoreType.DMA,
        pltpu.SemaphoreType.DMA,
        pltpu.SemaphoreType.DMA,
        pltpu.SemaphoreType.DMA,
    ],
    compiler_params=pltpu.CompilerParams(use_tc_tiling_on_sc=False),
)
def _gather(idxf, wt, inter, ixv, g0, g1, sg0, sg1, sw0, sw1):
    wid = lax.axis_index("s") * NC + lax.axis_index("c")
    base = wid * RPW
    pltpu.sync_copy(idxf.at[pl.ds(base, RPW)], ixv)

    def g_copy(g, gv, sem):
        return pltpu.make_async_copy(
            wt.at[ixv.at[pl.ds(g * GR, GR)]], gv, sem
        )

    def w_copy(g, gv, sem):
        return pltpu.make_async_copy(
            gv, inter.at[pl.ds(base + g * GR, GR)], sem
        )

    g_copy(0, g0, sg0).start()

    def body(g, _):
        @pl.when(lax.rem(g, 2) == 0)
        def _():
            @pl.when(g + 1 < NG)
            def _():
                @pl.when(g >= 1)
                def _():
                    w_copy(g - 1, g1, sw1).wait()

                g_copy(g + 1, g1, sg1).start()

            g_copy(g, g0, sg0).wait()
            w_copy(g, g0, sw0).start()

        @pl.when(lax.rem(g, 2) == 1)
        def _():
            @pl.when(g + 1 < NG)
            def _():
                w_copy(g - 1, g0, sw0).wait()
                g_copy(g + 1, g0, sg0).start()

            g_copy(g, g1, sg1).wait()
            w_copy(g, g1, sw1).start()

        return ()

    lax.fori_loop(0, NG, body, ())
    w_copy(NG - 2, g0, sw0).wait()
    w_copy(NG - 1, g1, sw1).wait()


RCH = 512
NCH = NI // (4 * RCH)


def _tc_body(in_ref, out_ref):
    x = in_ref[...]
    xt = x.T.reshape(4, D, RCH)
    out_ref[...] = xt.transpose(1, 0, 2).reshape(1, D, 4 * RCH)


_transpose = pl.pallas_call(
    _tc_body,
    out_shape=jax.ShapeDtypeStruct((NJ, D, NI), jnp.float32),
    grid=(NJ, NCH),
    in_specs=[pl.BlockSpec((RCH, 128), lambda j, t: (j * NCH + t, 0))],
    out_specs=pl.BlockSpec((1, D, 4 * RCH), lambda j, t: (j, 0, t)),
)


def kernel(idx, weight):
    idxp = (
        idx.T.reshape(NJ, NCH, 4, RCH)
        .transpose(0, 1, 3, 2)
        .reshape(-1)
        .astype(jnp.int32)
    )
    inter = _gather(idxp, weight)
    o3 = _transpose(inter.reshape(B // 4, 128))
    return o3.transpose(2, 0, 1)
